# Initial kernel scaffold; baseline (speedup 1.0000x reference)
import jax
import jax.numpy as jnp
from jax.experimental import pallas as pl


def kernel(x, W1, g1, b1, W2, g2, b2, W3, g3, b3, W4, g4, b4, W5, g5, b5):
    raise NotImplementedError("write your pallas kernel here")


# speedup vs baseline: 7.9543x; 7.9543x over previous
import functools

import jax
import jax.numpy as jnp
from jax import lax
from jax.experimental import pallas as pl
from jax.experimental.pallas import tpu as pltpu
from jax.experimental.pallas import tpu_sc as plsc

KNN = 20
BB = 8
NN = 1024
ROWS = BB * NN
CP = 128
NC, NS = 2, 16
NW = NC * NS
RPW = ROWS // NW
GROWS = 2 * KNN
GFIRE = 8
GNODES = 2 * GFIRE
NGRP = RPW // GNODES
CNT = float(BB * NN * KNN)
EPS = 1e-5
TT = 128
NT = NN // TT


def _lrelu(v):
    return jnp.where(v > 0, v, 0.2 * v)


def _bn_affine(h, mean, var, g, b):
    xn = g[None, :] * (h - mean[None, :]) / jnp.sqrt(var + EPS)[None, :] + b[None, :]
    return _lrelu(xn)


def _knn_topk(xt, idx_ref, dscr):
    b = pl.program_id(0)
    p = xt[:, :3]
    sq = jnp.sum(p * p, axis=1)
    G = lax.dot_general(p, p, (((1,), (1,)), ((), ())),
                        preferred_element_type=jnp.float32)
    dscr[...] = 2.0 * G - sq[:, None] - sq[None, :]
    colio = lax.broadcasted_iota(jnp.int32, (NN, NN), 1)
    ams = []
    for _ in range(KNN):
        d = dscr[...]
        m = jnp.max(d, axis=1)
        am = jnp.min(jnp.where(d == m[:, None], colio, NN), axis=1)
        ams.append(am)
        dscr[...] = jnp.where(colio == am[:, None], -jnp.inf, d)
    idx_ref[0] = jnp.stack(ams, axis=1) + b * NN


def _pad_cp(xt):
    c = xt.shape[1]
    if c == CP:
        return xt
    return jnp.concatenate([xt, jnp.zeros((xt.shape[0], CP - c), xt.dtype)],
                           axis=1)


def _k1_first_body(xt_ref, idx_ref, xout_ref, dscr):
    xt = xt_ref[0]
    xout_ref[0] = _pad_cp(xt)
    _knn_topk(xt, idx_ref, dscr)


def _k1_next_body(h_ref, m_ref, v_ref, g_ref, b_ref, idx_ref, xout_ref, dscr):
    xt = _bn_affine(h_ref[0], m_ref[...], v_ref[...], g_ref[...], b_ref[...])
    xout_ref[0] = _pad_cp(xt)
    _knn_topk(xt, idx_ref, dscr)


def _row_spec(c):
    return pl.BlockSpec((1, NN, c), lambda b: (b, 0, 0))


def _full_spec(shape):
    nd = len(shape)
    return pl.BlockSpec(shape, lambda b: (0,) * nd)


def _k1_first(xt):
    cin = xt.shape[-1]
    return pl.pallas_call(
        _k1_first_body,
        grid=(BB,),
        in_specs=[_row_spec(cin)],
        out_specs=[_row_spec(KNN), _row_spec(CP)],
        out_shape=[
            jax.ShapeDtypeStruct((BB, NN, KNN), jnp.int32),
            jax.ShapeDtypeStruct((BB, NN, CP), jnp.float32),
        ],
        scratch_shapes=[pltpu.VMEM((NN, NN), jnp.float32)],
    )(xt)


def _k1_next(h, m, v, g, b):
    cin = h.shape[-1]
    return pl.pallas_call(
        _k1_next_body,
        grid=(BB,),
        in_specs=[
            _row_spec(cin), _full_spec(m.shape), _full_spec(v.shape),
            _full_spec(g.shape), _full_spec(b.shape),
        ],
        out_specs=[_row_spec(KNN), _row_spec(CP)],
        out_shape=[
            jax.ShapeDtypeStruct((BB, NN, KNN), jnp.int32),
            jax.ShapeDtypeStruct((BB, NN, CP), jnp.float32),
        ],
        scratch_shapes=[pltpu.VMEM((NN, NN), jnp.float32)],
    )(h, m, v, g, b)


@functools.lru_cache(maxsize=None)
def _sc_feat():
    mesh = plsc.VectorSubcoreMesh(core_axis_name="c", subcore_axis_name="s")

    @functools.partial(
        pl.kernel,
        out_type=jax.ShapeDtypeStruct((ROWS * KNN, CP), jnp.float32),
        mesh=mesh,
        scratch_types=[
            pltpu.VMEM((RPW * KNN,), jnp.int32),
            pltpu.VMEM((GFIRE * GROWS, CP), jnp.float32),
            pltpu.SemaphoreType.DMA,
        ],
    )
    def k(xtab_hbm, idx_hbm, feat_hbm, idx_v, gb, gsem):
        wid = lax.axis_index("s") * NC + lax.axis_index("c")
        ebase = wid * RPW * KNN
        pltpu.sync_copy(idx_hbm.at[pl.ds(ebase, RPW * KNN)], idx_v)

        def group(g, _):
            for pj in range(GFIRE):
                off = pl.multiple_of((g * GFIRE + pj) * GROWS, 8)
                pltpu.make_async_copy(
                    xtab_hbm.at[idx_v.at[pl.ds(off, GROWS)]],
                    gb.at[pl.ds(pj * GROWS, GROWS)], gsem,
                ).start()
            for pj in range(GFIRE):
                pltpu.make_async_copy(
                    xtab_hbm.at[idx_v.at[pl.ds(0, GROWS)]],
                    gb.at[pl.ds(pj * GROWS, GROWS)], gsem,
                ).wait()
            pltpu.sync_copy(
                gb,
                feat_hbm.at[pl.ds(ebase + g * (GFIRE * GROWS), GFIRE * GROWS)],
            )
            return 0

        lax.fori_loop(0, NGRP, group, 0)

    return k


def _k3_body(feat_ref, xtab_ref, w_ref, hmax_ref, hfull_ref):
    o, c2 = w_ref.shape
    c = c2 // 2
    f3 = feat_ref[0].reshape(TT, KNN, CP)
    xc = xtab_ref[0]
    df = f3 - xc[:, None, :]
    xcr = jnp.broadcast_to(xc[:, None, :c], (TT, KNN, c))
    eg = jnp.concatenate([df[:, :, :c], xcr], axis=2).reshape(TT * KNN, c2)
    hh = lax.dot_general(eg, w_ref[...], (((1,), (1,)), ((), ())),
                         preferred_element_type=jnp.float32)
    h3 = hh.reshape(TT, KNN, o)
    m = h3[:, 0, :]
    for kk in range(1, KNN):
        m = jnp.maximum(m, h3[:, kk, :])
    hmax_ref[0] = m
    hfull_ref[0] = hh


def _k3_conv(feat, xtab, w):
    o = w.shape[0]
    return pl.pallas_call(
        _k3_body,
        grid=(BB, NT),
        in_specs=[
            pl.BlockSpec((1, TT * KNN, CP), lambda b, t: (b, t, 0)),
            pl.BlockSpec((1, TT, CP), lambda b, t: (b, t, 0)),
            pl.BlockSpec(w.shape, lambda b, t: (0, 0)),
        ],
        out_specs=[
            pl.BlockSpec((1, TT, o), lambda b, t: (b, t, 0)),
            pl.BlockSpec((1, TT * KNN, o), lambda b, t: (b, t, 0)),
        ],
        out_shape=[
            jax.ShapeDtypeStruct((BB, NN, o), jnp.float32),
            jax.ShapeDtypeStruct((BB, NN * KNN, o), jnp.float32),
        ],
    )(feat, xtab, w)


def _k3a_body(x1_ref, x2_ref, x3_ref, h4_ref, m4_ref, v4_ref, g_ref, b_ref,
              w5_ref, raw_ref):
    x4 = _bn_affine(h4_ref[0], m4_ref[...], v4_ref[...], g_ref[...], b_ref[...])
    xcat = jnp.concatenate(
        [x1_ref[0][:, :64], x2_ref[0][:, :64], x3_ref[0][:, :128], x4], axis=1)
    raw = lax.dot_general(w5_ref[...], xcat, (((1,), (1,)), ((), ())),
                          preferred_element_type=jnp.float32)
    raw_ref[0] = raw


def _k3c_body(raw_ref, m_ref, v_ref, g_ref, b_ref, out_ref):
    mean = m_ref[...]
    var = v_ref[...]
    o = (g_ref[...][:, None] * (raw_ref[0] - mean[:, None])
         / jnp.sqrt(var + EPS)[:, None] + b_ref[...][:, None])
    out_ref[0] = _lrelu(o)


def _stats(hfull):
    ht = jnp.transpose(hfull.reshape(BB, NN, KNN, hfull.shape[-1]),
                       (0, 3, 1, 2))
    ht = lax.optimization_barrier(ht)
    return jnp.mean(ht, axis=(0, 2, 3)), jnp.var(ht, axis=(0, 2, 3))


def kernel(x, W1, g1, b1, W2, g2, b2, W3, g3, b3, W4, g4, b4, W5, g5, b5):
    xt0 = jnp.transpose(x, (0, 2, 1))

    sc = _sc_feat()

    idx1, xtab0 = _k1_first(xt0)
    feat1 = sc(xtab0.reshape(ROWS, CP), idx1.reshape(ROWS * KNN))
    h1, hf1 = _k3_conv(feat1.reshape(BB, NN * KNN, CP), xtab0, W1)
    m1, v1 = _stats(hf1)

    idx2, xtab1 = _k1_next(h1, m1, v1, g1, b1)
    feat2 = sc(xtab1.reshape(ROWS, CP), idx2.reshape(ROWS * KNN))
    h2, hf2 = _k3_conv(feat2.reshape(BB, NN * KNN, CP), xtab1, W2)
    m2, v2 = _stats(hf2)

    idx3, xtab2 = _k1_next(h2, m2, v2, g2, b2)
    feat3 = sc(xtab2.reshape(ROWS, CP), idx3.reshape(ROWS * KNN))
    h3, hf3 = _k3_conv(feat3.reshape(BB, NN * KNN, CP), xtab2, W3)
    m3, v3 = _stats(hf3)

    idx4, xtab3 = _k1_next(h3, m3, v3, g3, b3)
    feat4 = sc(xtab3.reshape(ROWS, CP), idx4.reshape(ROWS * KNN))
    h4, hf4 = _k3_conv(feat4.reshape(BB, NN * KNN, CP), xtab3, W4)
    m4, v4 = _stats(hf4)

    raw = pl.pallas_call(
        _k3a_body,
        grid=(BB,),
        in_specs=[
            _row_spec(CP), _row_spec(CP), _row_spec(CP), _row_spec(256),
            _full_spec(m4.shape), _full_spec(v4.shape),
            _full_spec(g4.shape), _full_spec(b4.shape),
            _full_spec(W5.shape),
        ],
        out_specs=pl.BlockSpec((1, 1024, NN), lambda b: (b, 0, 0)),
        out_shape=jax.ShapeDtypeStruct((BB, 1024, NN), jnp.float32),
    )(xtab1, xtab2, xtab3, h4, m4, v4, g4, b4, W5)

    raw = lax.optimization_barrier(raw)
    m5 = jnp.mean(raw, axis=(0, 2))
    v5 = jnp.var(raw, axis=(0, 2))
    out = pl.pallas_call(
        _k3c_body,
        grid=(BB,),
        in_specs=[
            pl.BlockSpec((1, 1024, NN), lambda b: (b, 0, 0)),
            _full_spec(m5.shape), _full_spec(v5.shape),
            _full_spec(g5.shape), _full_spec(b5.shape),
        ],
        out_specs=pl.BlockSpec((1, 1024, NN), lambda b: (b, 0, 0)),
        out_shape=jax.ShapeDtypeStruct((BB, 1024, NN), jnp.float32),
    )(raw, m5, v5, g5, b5)
    return out
